# baseline (speedup 1.0000x reference)
import functools

import jax
import jax.numpy as jnp
from jax import lax
from jax.experimental import pallas as pl
from jax.experimental.pallas import tpu as pltpu
from jax.experimental.pallas import tpu_sc as plsc

_D = 128
_E = 320000
_T = 64
_SUM_ROW = 60
_ZERO_ROW = 61

_L = 16
_NW = 32
_PER_W = _E // _NW

_G = 40
_SLOTS = 5
_SET = _SLOTS * _G
_SUPER = 2 * _SET
_NSUPER = _PER_W // _SUPER


def _sc_kernel(ea, table0, table1, table2, summary):
    info = plsc.get_sparse_core_info()
    nc = info.num_cores
    mesh = plsc.VectorSubcoreMesh(core_axis_name="c", subcore_axis_name="s")

    @functools.partial(
        pl.kernel,
        out_type=jax.ShapeDtypeStruct((_E, _D), jnp.float32),
        mesh=mesh,
        scratch_types=[
            pltpu.VMEM((_PER_W,), jnp.int32),
            pltpu.VMEM((_PER_W,), jnp.int32),
            pltpu.VMEM((_PER_W,), jnp.int32),
            pltpu.VMEM((_PER_W,), jnp.int32),
            pltpu.VMEM((_SUPER, _D), jnp.float32),
            pltpu.VMEM((_T, _D), jnp.float32),
            pltpu.VMEM((5, _D), jnp.float32),
            pltpu.VMEM((6, _D), jnp.float32),
            pltpu.VMEM((2, _D), jnp.float32),
            pltpu.VMEM((1, _D), jnp.float32),
            pltpu.VMEM_SHARED((_T, _D), jnp.float32),
            pltpu.SemaphoreType.DMA,
            pltpu.SemaphoreType.DMA,
            pltpu.SemaphoreType.DMA,
            pltpu.SemaphoreType.DMA,
        ],
    )
    def body(ea_hbm, t0_hbm, t1_hbm, t2_hbm, su_hbm, out_hbm,
             col0, col1, col2, idxf, rows, combv, t0v, t1v, t2v, suv,
             comb_sh, gsA, gsB, wsA, wsB):
        sid = lax.axis_index("s")
        wid = sid * nc + lax.axis_index("c")
        base = wid * _PER_W

        ccol = pltpu.make_async_copy(ea_hbm.at[pl.ds(base, _PER_W)],
                                     col0, wsA)
        ccol.start()
        ccol1 = pltpu.make_async_copy(ea_hbm.at[pl.ds(_E + base, _PER_W)],
                                      col1, wsA)
        ccol1.start()
        ccol2 = pltpu.make_async_copy(ea_hbm.at[pl.ds(2 * _E + base, _PER_W)],
                                      col2, wsA)
        ccol2.start()

        pltpu.sync_copy(t0_hbm, t0v)
        pltpu.sync_copy(t1_hbm, t1v)
        pltpu.sync_copy(t2_hbm, t2v)
        pltpu.sync_copy(su_hbm, suv)

        def combo(r, carry):
            i0 = r // 12
            i1 = (r // 2) % 6
            i2 = r % 2
            for c in range(_D // _L):
                sl = pl.ds(c * _L, _L)
                combv[r, sl] = t0v[i0, sl] + t1v[i1, sl] + t2v[i2, sl]
            return carry

        lax.fori_loop(0, 60, combo, 0)
        zeros = jnp.zeros((_L,), jnp.float32)
        for c in range(_D // _L):
            sl = pl.ds(c * _L, _L)
            combv[_SUM_ROW, sl] = suv[0, sl]
            combv[_ZERO_ROW, sl] = zeros
            combv[_ZERO_ROW + 1, sl] = zeros
            combv[_ZERO_ROW + 2, sl] = zeros

        @pl.when(sid == 0)
        def _():
            pltpu.sync_copy(combv, comb_sh)

        ccol.wait()
        ccol1.wait()
        ccol2.wait()
        plsc.subcore_barrier()

        def compute_idx(t):
            def grp(r, carry):
                o = t * _SUPER + r * _L
                a = col0[pl.ds(o, _L)]
                b = col1[pl.ds(o, _L)]
                c = col2[pl.ds(o, _L)]
                s = a + b + c
                idx_n = (jnp.clip(a, 0, 4) * 12 + jnp.clip(b, 0, 5) * 2
                         + jnp.clip(c, 0, 1))
                idxf[pl.ds(o, _L)] = jnp.where(
                    s >= 0, idx_n,
                    jnp.where(s == -3,
                              jnp.full((_L,), _SUM_ROW, jnp.int32),
                              jnp.full((_L,), _ZERO_ROW, jnp.int32)))
                return carry

            lax.fori_loop(0, _SUPER // _L, grp, 0)

        def g_copy(set_off, row0, fire):
            for b in range(_SLOTS):
                src = comb_sh.at[idxf.at[pl.ds((row0 - base) + b * _G, _G)]]
                dst = rows.at[pl.ds(set_off + b * _G, _G)]
                sem = gsA if set_off == 0 else gsB
                cp = pltpu.make_async_copy(src, dst, sem)
                cp.start() if fire else cp.wait()

        def w_copy(set_off, row0, fire):
            sem = wsA if set_off == 0 else wsB
            for off, n in ((0, 120), (120, 80)):
                cp = pltpu.make_async_copy(
                    rows.at[pl.ds(set_off + off, n)],
                    out_hbm.at[pl.ds(row0 + off, n)], sem)
                cp.start() if fire else cp.wait()

        def superchunk(t, first=False, last=False):
            rA = base + t * _SUPER
            rB = rA + _SET
            g_copy(0, rA, fire=False)
            if not first:
                w_copy(_SET, rB, fire=False)
            g_copy(_SET, rB, fire=True)
            w_copy(0, rA, fire=True)
            if not last:
                compute_idx(t + 1)
            g_copy(_SET, rB, fire=False)
            w_copy(0, rA, fire=False)
            if not last:
                g_copy(0, rA + _SUPER, fire=True)
            w_copy(_SET, rB, fire=True)
            if last:
                w_copy(_SET, rB, fire=False)

        compute_idx(0)
        g_copy(0, base, fire=True)

        superchunk(0, first=True)

        def mid(t, carry):
            superchunk(t)
            return carry

        lax.fori_loop(1, _NSUPER - 1, mid, 0)
        superchunk(_NSUPER - 1, last=True)

    return body(ea, table0, table1, table2, summary)


def kernel(edge_attr, table0, table1, table2, summary):
    ea_t = edge_attr.astype(jnp.int32).T.reshape(-1)
    return _sc_kernel(ea_t, table0, table1, table2, summary)


# speedup vs baseline: 1.0149x; 1.0149x over previous
import functools

import jax
import jax.numpy as jnp
from jax import lax
from jax.experimental import pallas as pl
from jax.experimental.pallas import tpu as pltpu
from jax.experimental.pallas import tpu_sc as plsc

_D = 128
_E = 320000
_T = 64
_SUM_ROW = 60
_ZERO_ROW = 61

_L = 16
_NW = 32
_PER_W = _E // _NW

_G = 40
_SLOTS = 5
_SET = _SLOTS * _G
_SUPER = 2 * _SET
_NSUPER = _PER_W // _SUPER


def _sc_kernel(ea, table0, table1, table2, summary):
    info = plsc.get_sparse_core_info()
    nc = info.num_cores
    mesh = plsc.VectorSubcoreMesh(core_axis_name="c", subcore_axis_name="s")

    @functools.partial(
        pl.kernel,
        out_type=jax.ShapeDtypeStruct((_E, _D), jnp.float32),
        mesh=mesh,
        scratch_types=[
            pltpu.VMEM((_PER_W,), jnp.int32),
            pltpu.VMEM((_PER_W,), jnp.int32),
            pltpu.VMEM((_PER_W,), jnp.int32),
            pltpu.VMEM((_PER_W,), jnp.int32),
            pltpu.VMEM((_SUPER, _D), jnp.float32),
            pltpu.VMEM((_T, _D), jnp.float32),
            pltpu.VMEM((5, _D), jnp.float32),
            pltpu.VMEM((6, _D), jnp.float32),
            pltpu.VMEM((2, _D), jnp.float32),
            pltpu.VMEM((1, _D), jnp.float32),
            pltpu.VMEM_SHARED((_T, _D), jnp.float32),
            pltpu.SemaphoreType.DMA,
            pltpu.SemaphoreType.DMA,
            pltpu.SemaphoreType.DMA,
            pltpu.SemaphoreType.DMA,
        ],
    )
    def body(ea_hbm, t0_hbm, t1_hbm, t2_hbm, su_hbm, out_hbm,
             col0, col1, col2, idxf, rows, combv, t0v, t1v, t2v, suv,
             comb_sh, gsA, gsB, wsA, wsB):
        sid = lax.axis_index("s")
        wid = sid * nc + lax.axis_index("c")
        base = wid * _PER_W

        ccol = pltpu.make_async_copy(ea_hbm.at[pl.ds(base, _PER_W)],
                                     col0, wsA)
        ccol.start()
        ccol1 = pltpu.make_async_copy(ea_hbm.at[pl.ds(_E + base, _PER_W)],
                                      col1, wsA)
        ccol1.start()
        ccol2 = pltpu.make_async_copy(ea_hbm.at[pl.ds(2 * _E + base, _PER_W)],
                                      col2, wsA)
        ccol2.start()

        pltpu.sync_copy(t0_hbm, t0v)
        pltpu.sync_copy(t1_hbm, t1v)
        pltpu.sync_copy(t2_hbm, t2v)
        pltpu.sync_copy(su_hbm, suv)

        def combo(r, carry):
            i0 = r // 12
            i1 = (r // 2) % 6
            i2 = r % 2
            for c in range(_D // _L):
                sl = pl.ds(c * _L, _L)
                combv[r, sl] = t0v[i0, sl] + t1v[i1, sl] + t2v[i2, sl]
            return carry

        lax.fori_loop(0, 60, combo, 0)
        zeros = jnp.zeros((_L,), jnp.float32)
        for c in range(_D // _L):
            sl = pl.ds(c * _L, _L)
            combv[_SUM_ROW, sl] = suv[0, sl]
            combv[_ZERO_ROW, sl] = zeros
            combv[_ZERO_ROW + 1, sl] = zeros
            combv[_ZERO_ROW + 2, sl] = zeros

        @pl.when(sid == 0)
        def _():
            pltpu.sync_copy(combv, comb_sh)

        ccol.wait()
        ccol1.wait()
        ccol2.wait()
        plsc.subcore_barrier()

        def compute_idx(t):
            def grp(r, carry):
                o = t * _SUPER + r * _L
                a = col0[pl.ds(o, _L)]
                b = col1[pl.ds(o, _L)]
                c = col2[pl.ds(o, _L)]
                s = a + b + c
                idx_n = (jnp.clip(a, 0, 4) * 12 + jnp.clip(b, 0, 5) * 2
                         + jnp.clip(c, 0, 1))
                idxf[pl.ds(o, _L)] = jnp.where(
                    s >= 0, idx_n,
                    jnp.where(s == -3,
                              jnp.full((_L,), _SUM_ROW, jnp.int32),
                              jnp.full((_L,), _ZERO_ROW, jnp.int32)))
                return carry

            lax.fori_loop(0, _SUPER // _L, grp, 0)

        def g_copy(set_off, row0, fire):
            for b in range(_SLOTS):
                src = comb_sh.at[idxf.at[pl.ds((row0 - base) + b * _G, _G)]]
                dst = rows.at[pl.ds(set_off + b * _G, _G)]
                sem = gsA if set_off == 0 else gsB
                cp = pltpu.make_async_copy(src, dst, sem)
                cp.start() if fire else cp.wait()

        def w_copy(set_off, row0, fire):
            sem = wsA if set_off == 0 else wsB
            cp = pltpu.make_async_copy(
                rows.at[pl.ds(set_off, _SET)],
                out_hbm.at[pl.ds(row0, _SET)], sem)
            cp.start() if fire else cp.wait()

        def superchunk(t, first=False, last=False):
            rA = base + t * _SUPER
            rB = rA + _SET
            g_copy(0, rA, fire=False)
            if not first:
                w_copy(_SET, rB, fire=False)
            g_copy(_SET, rB, fire=True)
            w_copy(0, rA, fire=True)
            if not last:
                compute_idx(t + 1)
            w_copy(0, rA, fire=False)
            if not last:
                g_copy(0, rA + _SUPER, fire=True)
            g_copy(_SET, rB, fire=False)
            w_copy(_SET, rB, fire=True)
            if last:
                w_copy(_SET, rB, fire=False)

        compute_idx(0)
        g_copy(0, base, fire=True)

        superchunk(0, first=True)

        def mid(t, carry):
            superchunk(t)
            return carry

        lax.fori_loop(1, _NSUPER - 1, mid, 0)
        superchunk(_NSUPER - 1, last=True)

    return body(ea, table0, table1, table2, summary)


def kernel(edge_attr, table0, table1, table2, summary):
    ea_t = edge_attr.astype(jnp.int32).T.reshape(-1)
    return _sc_kernel(ea_t, table0, table1, table2, summary)


# speedup vs baseline: 1.0383x; 1.0230x over previous
import functools

import jax
import jax.numpy as jnp
from jax import lax
from jax.experimental import pallas as pl
from jax.experimental.pallas import tpu as pltpu
from jax.experimental.pallas import tpu_sc as plsc

_D = 128
_E = 320000
_T = 64
_SUM_ROW = 60
_ZERO_ROW = 61

_L = 16
_NW = 32
_PER_W = _E // _NW

_G = 40
_SLOTS = 5
_SET = _SLOTS * _G
_SUPER = 2 * _SET
_NSUPER = _PER_W // _SUPER


def _sc_kernel(ea, table0, table1, table2, summary):
    info = plsc.get_sparse_core_info()
    nc = info.num_cores
    mesh = plsc.VectorSubcoreMesh(core_axis_name="c", subcore_axis_name="s")

    @functools.partial(
        pl.kernel,
        out_type=jax.ShapeDtypeStruct((_E, _D), jnp.float32),
        mesh=mesh,
        scratch_types=[
            pltpu.VMEM((_PER_W,), jnp.int32),
            pltpu.VMEM((_PER_W,), jnp.int32),
            pltpu.VMEM((_PER_W,), jnp.int32),
            pltpu.VMEM((_PER_W,), jnp.int32),
            pltpu.VMEM((_SUPER, _D), jnp.float32),
            pltpu.VMEM((_T, _D), jnp.float32),
            pltpu.VMEM((5, _D), jnp.float32),
            pltpu.VMEM((6, _D), jnp.float32),
            pltpu.VMEM((2, _D), jnp.float32),
            pltpu.VMEM((1, _D), jnp.float32),
            pltpu.VMEM_SHARED((_T, _D), jnp.float32),
            pltpu.SemaphoreType.DMA,
            pltpu.SemaphoreType.DMA,
            pltpu.SemaphoreType.DMA,
            pltpu.SemaphoreType.DMA,
        ],
    )
    def body(ea_hbm, t0_hbm, t1_hbm, t2_hbm, su_hbm, out_hbm,
             col0, col1, col2, idxf, rows, combv, t0v, t1v, t2v, suv,
             comb_sh, gsA, gsB, wsA, wsB):
        sid = lax.axis_index("s")
        wid = sid * nc + lax.axis_index("c")
        base = wid * _PER_W

        ccol = pltpu.make_async_copy(ea_hbm.at[pl.ds(base, _PER_W)],
                                     col0, wsA)
        ccol.start()
        ccol1 = pltpu.make_async_copy(ea_hbm.at[pl.ds(_E + base, _PER_W)],
                                      col1, wsA)
        ccol1.start()
        ccol2 = pltpu.make_async_copy(ea_hbm.at[pl.ds(2 * _E + base, _PER_W)],
                                      col2, wsA)
        ccol2.start()

        @pl.when(sid == 0)
        def _():
            pltpu.sync_copy(t0_hbm, t0v)
            pltpu.sync_copy(t1_hbm, t1v)
            pltpu.sync_copy(t2_hbm, t2v)
            pltpu.sync_copy(su_hbm, suv)

            def combo(r, carry):
                i0 = r // 12
                i1 = (r // 2) % 6
                i2 = r % 2
                for c in range(_D // _L):
                    sl = pl.ds(c * _L, _L)
                    combv[r, sl] = t0v[i0, sl] + t1v[i1, sl] + t2v[i2, sl]
                return carry

            lax.fori_loop(0, 60, combo, 0)
            zeros = jnp.zeros((_L,), jnp.float32)
            for c in range(_D // _L):
                sl = pl.ds(c * _L, _L)
                combv[_SUM_ROW, sl] = suv[0, sl]
                combv[_ZERO_ROW, sl] = zeros
                combv[_ZERO_ROW + 1, sl] = zeros
                combv[_ZERO_ROW + 2, sl] = zeros
            pltpu.sync_copy(combv, comb_sh)

        ccol.wait()
        ccol1.wait()
        ccol2.wait()

        def compute_idx(t):
            def grp(r, carry):
                o = t * _SUPER + r * _L
                a = col0[pl.ds(o, _L)]
                b = col1[pl.ds(o, _L)]
                c = col2[pl.ds(o, _L)]
                s = a + b + c
                idx_n = (jnp.clip(a, 0, 4) * 12 + jnp.clip(b, 0, 5) * 2
                         + jnp.clip(c, 0, 1))
                idxf[pl.ds(o, _L)] = jnp.where(
                    s >= 0, idx_n,
                    jnp.where(s == -3,
                              jnp.full((_L,), _SUM_ROW, jnp.int32),
                              jnp.full((_L,), _ZERO_ROW, jnp.int32)))
                return carry

            lax.fori_loop(0, _SUPER // _L, grp, 0)

        def g_copy(set_off, row0, fire):
            for b in range(_SLOTS):
                src = comb_sh.at[idxf.at[pl.ds((row0 - base) + b * _G, _G)]]
                dst = rows.at[pl.ds(set_off + b * _G, _G)]
                sem = gsA if set_off == 0 else gsB
                cp = pltpu.make_async_copy(src, dst, sem)
                cp.start() if fire else cp.wait()

        def w_copy(set_off, row0, fire):
            sem = wsA if set_off == 0 else wsB
            cp = pltpu.make_async_copy(
                rows.at[pl.ds(set_off, _SET)],
                out_hbm.at[pl.ds(row0, _SET)], sem)
            cp.start() if fire else cp.wait()

        def superchunk(t, first=False, last=False):
            rA = base + t * _SUPER
            rB = rA + _SET
            g_copy(0, rA, fire=False)
            if not first:
                w_copy(_SET, rB, fire=False)
            g_copy(_SET, rB, fire=True)
            w_copy(0, rA, fire=True)
            if not last:
                compute_idx(t + 1)
            w_copy(0, rA, fire=False)
            if not last:
                g_copy(0, rA + _SUPER, fire=True)
            g_copy(_SET, rB, fire=False)
            w_copy(_SET, rB, fire=True)
            if last:
                w_copy(_SET, rB, fire=False)

        compute_idx(0)
        plsc.subcore_barrier()
        g_copy(0, base, fire=True)

        superchunk(0, first=True)

        def mid(t, carry):
            superchunk(t)
            return carry

        lax.fori_loop(1, _NSUPER - 1, mid, 0)
        superchunk(_NSUPER - 1, last=True)

    return body(ea, table0, table1, table2, summary)


def kernel(edge_attr, table0, table1, table2, summary):
    ea_t = edge_attr.astype(jnp.int32).T.reshape(-1)
    return _sc_kernel(ea_t, table0, table1, table2, summary)
